# baseline (speedup 1.0000x reference)
import functools
import math

import jax
import jax.numpy as jnp
import numpy as np
from jax import lax
from jax.experimental import pallas as pl
from jax.experimental.pallas import tpu as pltpu
from jax.experimental.pallas import tpu_sc as plsc

N = 10000
E = 160000
H = 64
R = 32
NCHUNK = 4
CW = H // NCHUNK
REC = 9 * CW
CUTOFF_UPPER = 4.5

NB = 1000
EB = 2000


def _build_D9R9():
    D9 = np.zeros((9, 9), np.float32)
    D9[0, 0] = D9[0, 4] = D9[0, 8] = 1.0 / 3.0
    D9[1, 1], D9[1, 3] = 0.5, -0.5
    D9[2, 2], D9[2, 6] = 0.5, -0.5
    D9[3, 5], D9[3, 7] = 0.5, -0.5
    D9[4, 0], D9[4, 4], D9[4, 8] = 2 / 3, -1 / 3, -1 / 3
    D9[5, 1] = D9[5, 3] = 0.5
    D9[6, 2] = D9[6, 6] = 0.5
    D9[7, 0], D9[7, 4], D9[7, 8] = -1 / 3, 2 / 3, -1 / 3
    D9[8, 5] = D9[8, 7] = 0.5
    R9 = np.zeros((9, 9), np.float32)
    R9[0, 0] = R9[0, 4] = 1
    R9[1, 1] = R9[1, 5] = 1
    R9[2, 2] = R9[2, 6] = 1
    R9[3, 1], R9[3, 5] = -1, 1
    R9[4, 0] = R9[4, 7] = 1
    R9[5, 3] = R9[5, 8] = 1
    R9[6, 2], R9[6, 6] = -1, 1
    R9[7, 3], R9[7, 8] = -1, 1
    R9[8, 0], R9[8, 4], R9[8, 7] = 1, -1, -1
    return D9, R9


_D9, _R9 = _build_D9R9()

_RecM = np.zeros((NCHUNK, 9, CW, 9, H), np.float32)
for _cq in range(NCHUNK):
    for _i in range(9):
        for _c in range(CW):
            _RecM[_cq, _i, _c, :, _cq * CW + _c] = _R9[:, _i]
_RecM = _RecM.reshape(576, 576)

_S64 = np.zeros((576, H), np.float32)
_E64 = np.zeros((H, 576), np.float32)
for _h in range(H):
    for _j in range(9):
        _S64[_h * 9 + _j, _h] = 1.0
        _E64[_h, _h * 9 + _j] = 1.0
_P2 = np.zeros((576, 576), np.float32)
for _h in range(H):
    for _j in range(9):
        _P2[_j * H + _h, _h * 9 + _j] = 1.0
_DEC = np.zeros((9, H, 9, H), np.float32)
for _i in range(9):
    for _j in range(9):
        _DEC[_j, :, _i, :] = _D9[_i, _j] * np.eye(H, dtype=np.float32)
_DEC = _DEC.reshape(9, H, 576).transpose(1, 0, 2).reshape(576, 576)


def _rows_to_hmajor(g):
    return g.reshape(9, H, g.shape[1]).transpose(1, 0, 2).reshape(576, g.shape[1])


def _mix_mats(w0, w1, w2):
    return jnp.stack([w0, w1, w1, w1, w2, w2, w2, w2, w2])


def _build_GY(wst):
    g = jnp.einsum('ij,ki,imh->jhkm', _D9, _R9, wst)
    return g.reshape(576, 576)


def _build_GT(wst):
    g = jnp.einsum('ij,idh->jhid', _D9, wst)
    g = g.reshape(9, H, 9, NCHUNK, CW).transpose(0, 1, 3, 2, 4)
    return g.reshape(576, 576)


def _edge_kernel(attr_ref, ew_ref, w1_ref, b1_ref, w2_ref, b2_ref, w3p_ref, b3p_ref, out_ref):
    def silu(x):
        return x * (1.0 / (1.0 + jnp.exp(-x)))
    a = attr_ref[...]
    h1 = silu(jnp.dot(a, w1_ref[...].T, preferred_element_type=jnp.float32) + b1_ref[...])
    h2 = silu(jnp.dot(h1, w2_ref[...].T, preferred_element_type=jnp.float32) + b2_ref[...])
    ew = ew_ref[...]
    cc = 0.5 * (jnp.cos(ew * (math.pi / CUTOFF_UPPER)) + 1.0)
    cc = jnp.where(ew < CUTOFF_UPPER, cc, 0.0)
    for cq in range(NCHUNK):
        o = silu(jnp.dot(h2, w3p_ref[cq].T, preferred_element_type=jnp.float32) + b3p_ref[cq])
        out_ref[cq] = o * cc


def _edge_mlp(edge_attr, edge_weight, ws1, bs1, ws2, bs2, ws3p, bs3p):
    nblk = E // EB
    return pl.pallas_call(
        _edge_kernel,
        grid=(nblk,),
        in_specs=[
            pl.BlockSpec((EB, R), lambda i: (i, 0)),
            pl.BlockSpec((EB, 1), lambda i: (i, 0)),
            pl.BlockSpec((H, R), lambda i: (0, 0)),
            pl.BlockSpec((1, H), lambda i: (0, 0)),
            pl.BlockSpec((2 * H, H), lambda i: (0, 0)),
            pl.BlockSpec((1, 2 * H), lambda i: (0, 0)),
            pl.BlockSpec((NCHUNK, 3 * CW, 2 * H), lambda i: (0, 0, 0)),
            pl.BlockSpec((NCHUNK, 1, 3 * CW), lambda i: (0, 0, 0)),
        ],
        out_specs=pl.BlockSpec((NCHUNK, EB, 3 * CW), lambda i: (0, i, 0)),
        out_shape=jax.ShapeDtypeStruct((NCHUNK, E, 3 * CW), jnp.float32),
    )(edge_attr, edge_weight.reshape(E, 1), ws1, bs1.reshape(1, H), ws2,
      bs2.reshape(1, 2 * H), ws3p, bs3p.reshape(NCHUNK, 1, 3 * CW))


def _prep_kernel(x_ref, gt_ref, gy_ref, s64_ref, e64_ref, t_ref, y_ref, xn_ref):
    x = x_ref[...]
    norm = jnp.dot(x * x, s64_ref[...], preferred_element_type=jnp.float32, precision=jax.lax.Precision.HIGHEST)
    inv = 1.0 / (norm + 1.0)
    invh = jnp.dot(inv, e64_ref[...], preferred_element_type=jnp.float32, precision=jax.lax.Precision.HIGHEST)
    xn = x * invh
    xn_ref[...] = xn
    t_ref[...] = jnp.dot(xn, gt_ref[...], preferred_element_type=jnp.float32)
    y_ref[...] = jnp.dot(xn, gy_ref[...], preferred_element_type=jnp.float32)


def _node_prep(xc, gt, gy, s64, e64):
    nblk = N // NB
    return pl.pallas_call(
        _prep_kernel,
        grid=(nblk,),
        in_specs=[
            pl.BlockSpec((NB, 576), lambda i: (i, 0)),
            pl.BlockSpec((576, 576), lambda i: (0, 0)),
            pl.BlockSpec((576, 576), lambda i: (0, 0)),
            pl.BlockSpec((576, H), lambda i: (0, 0)),
            pl.BlockSpec((H, 576), lambda i: (0, 0)),
        ],
        out_specs=[
            pl.BlockSpec((NB, 576), lambda i: (i, 0)),
            pl.BlockSpec((NB, 576), lambda i: (i, 0)),
            pl.BlockSpec((NB, 576), lambda i: (i, 0)),
        ],
        out_shape=[
            jax.ShapeDtypeStruct((N, 576), jnp.float32),
            jax.ShapeDtypeStruct((N, 576), jnp.float32),
            jax.ShapeDtypeStruct((N, 576), jnp.float32),
        ],
    )(xc, gt, gy, s64, e64)


_SC_TILES = 16
_B = 80
_EPT = E // _SC_TILES
_NB_E = _EPT // _B
_NPT = N // _SC_TILES


def _sc_msg_body(t_hbm, ea_hbm, src_hbm, dst_hbm, zero_hbm, out_hbm,
                 srcv, dstv, recv, eav, acc, sem_g, sem_e, sem_s):
    cid = lax.axis_index("c")
    sid = lax.axis_index("s")
    n0 = sid * _NPT
    e_base = sid * _EPT
    for cqi in range(2):
        cq = cid * 2 + cqi
        pltpu.sync_copy(zero_hbm.at[pl.ds(0, _NPT)], acc.at[pl.ds(n0, _NPT)])
        plsc.subcore_barrier()

        def issue(i, b):
            e0 = e_base + i * _B
            pltpu.sync_copy(src_hbm.at[pl.ds(e0, _B)], srcv.at[b])
            pltpu.sync_copy(dst_hbm.at[pl.ds(e0, _B)], dstv.at[b])
            for j in range(_B // CW):
                d = dstv[b, pl.ds(j * CW, CW)]
                dstv[b, pl.ds(j * CW, CW)] = d * NCHUNK + cq
            gcp = pltpu.async_copy(t_hbm.at[dstv.at[b]], recv.at[b], sem_g)
            ecp = pltpu.async_copy(ea_hbm.at[cq, pl.ds(e0, _B)], eav.at[b], sem_e)
            return gcp, ecp

        issue(0, 0)

        def batch(i, _):
            b = lax.rem(i, 2)
            bn = 1 - b
            pltpu.make_async_copy(t_hbm.at[dstv.at[b]], recv.at[b], sem_g).wait()
            pltpu.make_async_copy(ea_hbm.at[cq, pl.ds(0, _B)], eav.at[b], sem_e).wait()

            @pl.when(i > 0)
            def _():
                pltpu.make_async_copy(recv.at[bn], acc.at[srcv.at[bn]], sem_s).wait()

            @pl.when(i + 1 < _NB_E)
            def _():
                issue(i + 1, bn)

            def edge(e, _):
                w0 = eav[b, e, pl.ds(0, CW)]
                w1 = eav[b, e, pl.ds(CW, CW)]
                w2 = eav[b, e, pl.ds(2 * CW, CW)]
                recv[b, e, pl.ds(0, CW)] = w0 * recv[b, e, pl.ds(0, CW)]
                for j in (1, 2, 3):
                    recv[b, e, pl.ds(j * CW, CW)] = w1 * recv[b, e, pl.ds(j * CW, CW)]
                for j in (4, 5, 6, 7, 8):
                    recv[b, e, pl.ds(j * CW, CW)] = w2 * recv[b, e, pl.ds(j * CW, CW)]
                return 0

            lax.fori_loop(0, _B, edge, 0, unroll=4)
            pltpu.async_copy(recv.at[b], acc.at[srcv.at[b]], sem_s, add=True)
            return 0

        lax.fori_loop(0, _NB_E, batch, 0)
        blast = lax.rem(_NB_E - 1, 2)
        pltpu.make_async_copy(recv.at[blast], acc.at[srcv.at[blast]], sem_s).wait()
        plsc.subcore_barrier()
        pltpu.sync_copy(acc.at[pl.ds(n0, _NPT)], out_hbm.at[pl.ds(n0, _NPT), cq])
        plsc.subcore_barrier()


def _message_pass_sc(t_flat, ea4, src, dst, zeros):
    f = pl.kernel(
        _sc_msg_body,
        out_type=jax.ShapeDtypeStruct((N, NCHUNK, REC), jnp.float32),
        mesh=plsc.VectorSubcoreMesh(core_axis_name="c", subcore_axis_name="s",
                                    num_cores=2, num_subcores=_SC_TILES),
        scratch_types=[
            pltpu.VMEM((2, _B), jnp.int32),
            pltpu.VMEM((2, _B), jnp.int32),
            pltpu.VMEM((2, _B, REC), jnp.float32),
            pltpu.VMEM((2, _B, 3 * CW), jnp.float32),
            pltpu.VMEM_SHARED((N, REC), jnp.float32),
            pltpu.SemaphoreType.DMA,
            pltpu.SemaphoreType.DMA,
            pltpu.SemaphoreType.DMA,
        ],
        compiler_params=pltpu.CompilerParams(use_tc_tiling_on_sc=False),
    )
    return f(t_flat, ea4, src, dst, zeros).reshape(N, 576)


def _update_kernel(xn_ref, y_ref, mc_ref, q_ref, recm_ref, gd_ref, p2_ref, out_ref, t_ref):
    mp = jnp.dot(mc_ref[...], recm_ref[...], preferred_element_type=jnp.float32)
    y = y_ref[...]
    scale = 1.0 + 0.1 * q_ref[...]

    def pln(a, j):
        return a[:, j * H:(j + 1) * H]

    for r in range(3):
        for c in range(3):
            acc = jnp.zeros((mp.shape[0], H), jnp.float32)
            for k in range(3):
                acc += pln(mp, 3 * r + k) * pln(y, 3 * k + c)
                acc += pln(y, 3 * r + k) * pln(mp, 3 * k + c)
            t_ref[:, (3 * r + c) * H:(3 * r + c + 1) * H] = acc * scale
    dx = jnp.dot(t_ref[...], gd_ref[...], preferred_element_type=jnp.float32)
    dxhm = jnp.dot(dx, p2_ref[...], preferred_element_type=jnp.float32, precision=jax.lax.Precision.HIGHEST)
    for r in range(3):
        for c in range(3):
            acc = jnp.zeros((dx.shape[0], H), jnp.float32)
            for k in range(3):
                acc += pln(dx, 3 * r + k) * pln(dx, 3 * k + c)
            t_ref[:, (3 * r + c) * H:(3 * r + c + 1) * H] = acc
    ddxhm = jnp.dot(t_ref[...], p2_ref[...], preferred_element_type=jnp.float32, precision=jax.lax.Precision.HIGHEST)
    out_ref[...] = xn_ref[...] + dxhm + scale * ddxhm


def _node_update(xn, yc, mc, q2, recm, gd, p2):
    nblk = N // NB
    return pl.pallas_call(
        _update_kernel,
        grid=(nblk,),
        in_specs=[
            pl.BlockSpec((NB, 576), lambda i: (i, 0)),
            pl.BlockSpec((NB, 576), lambda i: (i, 0)),
            pl.BlockSpec((NB, 576), lambda i: (i, 0)),
            pl.BlockSpec((NB, 1), lambda i: (i, 0)),
            pl.BlockSpec((576, 576), lambda i: (0, 0)),
            pl.BlockSpec((576, 576), lambda i: (0, 0)),
            pl.BlockSpec((576, 576), lambda i: (0, 0)),
        ],
        out_specs=pl.BlockSpec((NB, 576), lambda i: (i, 0)),
        out_shape=jax.ShapeDtypeStruct((N, 576), jnp.float32),
        scratch_shapes=[pltpu.VMEM((NB, 576), jnp.float32)],
    )(xn, yc, mc, q2, recm, gd, p2)


def _post_kernel(x_ref, dec_ref, g_ref, b_ref, wp_ref, lb_ref, out_ref):
    x = jnp.dot(x_ref[...], dec_ref[...], preferred_element_type=jnp.float32, precision=jax.lax.Precision.HIGHEST)

    def pln(j):
        return x[:, j * H:(j + 1) * H]

    lam = pln(0)
    a0 = pln(1); a1 = pln(2); a2 = pln(3)
    s00 = pln(4); s01 = pln(5); s02 = pln(6)
    s11 = pln(7); s12 = pln(8); s22 = -s00 - s11
    nI = 3.0 * lam * lam
    nA = 2.0 * (a0 * a0 + a1 * a1 + a2 * a2)
    nS = s00 * s00 + s11 * s11 + s22 * s22 + 2.0 * (s01 * s01 + s02 * s02 + s12 * s12)
    ssum = (jnp.sum(nI, axis=1, keepdims=True) + jnp.sum(nA, axis=1, keepdims=True)
            + jnp.sum(nS, axis=1, keepdims=True))
    ssq = (jnp.sum(nI * nI, axis=1, keepdims=True) + jnp.sum(nA * nA, axis=1, keepdims=True)
           + jnp.sum(nS * nS, axis=1, keepdims=True))
    mean = ssum * (1.0 / (3 * H))
    var = ssq * (1.0 / (3 * H)) - mean * mean
    rstd = jax.lax.rsqrt(var + 1e-5)
    acc = jnp.zeros((x.shape[0], H), jnp.float32)
    for p, npart in enumerate((nI, nA, nS)):
        xi = (npart - mean) * rstd * g_ref[p].reshape(1, H) + b_ref[p].reshape(1, H)
        acc += jnp.dot(xi, wp_ref[p], preferred_element_type=jnp.float32)
    acc = acc + lb_ref[...]
    out_ref[...] = acc * (1.0 / (1.0 + jnp.exp(-acc)))


def _post(xc, dec, ln_g, ln_b, lin_W, lin_b):
    nblk = N // NB
    wp = jnp.stack([lin_W[:, p * H:(p + 1) * H].T for p in range(3)])
    return pl.pallas_call(
        _post_kernel,
        grid=(nblk,),
        in_specs=[
            pl.BlockSpec((NB, 576), lambda i: (i, 0)),
            pl.BlockSpec((576, 576), lambda i: (0, 0)),
            pl.BlockSpec((3, H), lambda i: (0, 0)),
            pl.BlockSpec((3, H), lambda i: (0, 0)),
            pl.BlockSpec((3, H, H), lambda i: (0, 0, 0)),
            pl.BlockSpec((1, H), lambda i: (0, 0)),
        ],
        out_specs=pl.BlockSpec((NB, H), lambda i: (i, 0)),
        out_shape=jax.ShapeDtypeStruct((N, H), jnp.float32),
    )(xc, dec, ln_g.reshape(3, H), ln_b.reshape(3, H), wp, lin_b.reshape(1, H))


def kernel(X, edge_index, edge_weight, edge_attr, q, ws1, bs1, ws2, bs2, ws3, bs3, wt,
           lin_W, lin_b, ln_g, ln_b):
    from jax._src import config as _jcfg
    with _jcfg.enable_x64(False):
        return _kernel_impl(X, edge_index, edge_weight, edge_attr, q, ws1, bs1, ws2, bs2,
                            ws3, bs3, wt, lin_W, lin_b, ln_g, ln_b)


def _kernel_impl(X, edge_index, edge_weight, edge_attr, q, ws1, bs1, ws2, bs2, ws3, bs3, wt,
                 lin_W, lin_b, ln_g, ln_b):
    L = ws1.shape[0]
    xc = X.reshape(N, 576)
    src = edge_index[0].astype(jnp.int32)
    dst = edge_index[1].astype(jnp.int32)
    zeros = jnp.zeros((_NPT, REC), jnp.float32)
    q2 = q.reshape(N, 1).astype(jnp.float32)
    recm = jnp.asarray(_RecM)
    p2 = jnp.asarray(_P2)
    s64 = jnp.asarray(_S64)
    e64 = jnp.asarray(_E64)
    dec = jnp.asarray(_DEC)

    for l in range(L):
        wst1 = _mix_mats(wt[l, 0], wt[l, 1], wt[l, 2])
        wst2 = _mix_mats(wt[l, 3], wt[l, 4], wt[l, 5])
        gt = _rows_to_hmajor(_build_GT(wst1))
        gy = _rows_to_hmajor(_build_GY(wst1))
        gd = _build_GY(wst2)
        ws3p = ws3[l].reshape(NCHUNK, CW, 3, 2 * H).transpose(0, 2, 1, 3).reshape(NCHUNK, 3 * CW, 2 * H)
        bs3p = bs3[l].reshape(NCHUNK, CW, 3).transpose(0, 2, 1).reshape(NCHUNK, 3 * CW)

        ea4 = _edge_mlp(edge_attr, edge_weight.astype(jnp.float32), ws1[l], bs1[l],
                        ws2[l], bs2[l], ws3p, bs3p)
        tc_tab, yc, xn = _node_prep(xc, gt, gy, s64, e64)
        mc = _message_pass_sc(tc_tab.reshape(N * NCHUNK, REC), ea4, src, dst, zeros)
        xc = _node_update(xn, yc, mc, q2, recm, gd, p2)

    post = _post(xc, dec, ln_g, ln_b, lin_W, lin_b)
    Xout = xc.reshape(N, H, 3, 3)
    return (Xout, post[:N - 1])


# speedup vs baseline: 1.1063x; 1.1063x over previous
import functools
import math

import jax
import jax.numpy as jnp
import numpy as np
from jax import lax
from jax.experimental import pallas as pl
from jax.experimental.pallas import tpu as pltpu
from jax.experimental.pallas import tpu_sc as plsc

N = 10000
E = 160000
H = 64
R = 32
NCHUNK = 4
CW = H // NCHUNK
REC = 9 * CW
CUTOFF_UPPER = 4.5

NB = 1000
EB = 2000


def _build_D9R9():
    D9 = np.zeros((9, 9), np.float32)
    D9[0, 0] = D9[0, 4] = D9[0, 8] = 1.0 / 3.0
    D9[1, 1], D9[1, 3] = 0.5, -0.5
    D9[2, 2], D9[2, 6] = 0.5, -0.5
    D9[3, 5], D9[3, 7] = 0.5, -0.5
    D9[4, 0], D9[4, 4], D9[4, 8] = 2 / 3, -1 / 3, -1 / 3
    D9[5, 1] = D9[5, 3] = 0.5
    D9[6, 2] = D9[6, 6] = 0.5
    D9[7, 0], D9[7, 4], D9[7, 8] = -1 / 3, 2 / 3, -1 / 3
    D9[8, 5] = D9[8, 7] = 0.5
    R9 = np.zeros((9, 9), np.float32)
    R9[0, 0] = R9[0, 4] = 1
    R9[1, 1] = R9[1, 5] = 1
    R9[2, 2] = R9[2, 6] = 1
    R9[3, 1], R9[3, 5] = -1, 1
    R9[4, 0] = R9[4, 7] = 1
    R9[5, 3] = R9[5, 8] = 1
    R9[6, 2], R9[6, 6] = -1, 1
    R9[7, 3], R9[7, 8] = -1, 1
    R9[8, 0], R9[8, 4], R9[8, 7] = 1, -1, -1
    return D9, R9


_D9, _R9 = _build_D9R9()

_RecM = np.zeros((NCHUNK, 9, CW, 9, H), np.float32)
for _cq in range(NCHUNK):
    for _i in range(9):
        for _c in range(CW):
            _RecM[_cq, _i, _c, :, _cq * CW + _c] = _R9[:, _i]
_RecM = _RecM.reshape(576, 576)

_S64 = np.zeros((576, H), np.float32)
_E64 = np.zeros((H, 576), np.float32)
for _h in range(H):
    for _j in range(9):
        _S64[_h * 9 + _j, _h] = 1.0
        _E64[_h, _h * 9 + _j] = 1.0
_P2 = np.zeros((576, 576), np.float32)
for _h in range(H):
    for _j in range(9):
        _P2[_j * H + _h, _h * 9 + _j] = 1.0
_DEC = np.zeros((9, H, 9, H), np.float32)
for _i in range(9):
    for _j in range(9):
        _DEC[_j, :, _i, :] = _D9[_i, _j] * np.eye(H, dtype=np.float32)
_DEC = _DEC.reshape(9, H, 576).transpose(1, 0, 2).reshape(576, 576)


def _rows_to_hmajor(g):
    return g.reshape(9, H, g.shape[1]).transpose(1, 0, 2).reshape(576, g.shape[1])


def _mix_mats(w0, w1, w2):
    return jnp.stack([w0, w1, w1, w1, w2, w2, w2, w2, w2])


def _build_GY(wst):
    g = jnp.einsum('ij,ki,imh->jhkm', _D9, _R9, wst)
    return g.reshape(576, 576)


def _build_GT(wst):
    g = jnp.einsum('ij,idh->jhid', _D9, wst)
    g = g.reshape(9, H, 9, NCHUNK, CW).transpose(0, 1, 3, 2, 4)
    return g.reshape(576, 576)


def _edge_kernel(attr_ref, ew_ref, w1_ref, b1_ref, w2_ref, b2_ref, w3p_ref, b3p_ref, out_ref):
    def silu(x):
        return x * (1.0 / (1.0 + jnp.exp(-x)))
    a = attr_ref[...]
    h1 = silu(jnp.dot(a, w1_ref[...].T, preferred_element_type=jnp.float32) + b1_ref[...])
    h2 = silu(jnp.dot(h1, w2_ref[...].T, preferred_element_type=jnp.float32) + b2_ref[...])
    ew = ew_ref[...]
    cc = 0.5 * (jnp.cos(ew * (math.pi / CUTOFF_UPPER)) + 1.0)
    cc = jnp.where(ew < CUTOFF_UPPER, cc, 0.0)
    for cq in range(NCHUNK):
        o = silu(jnp.dot(h2, w3p_ref[cq].T, preferred_element_type=jnp.float32) + b3p_ref[cq])
        out_ref[cq] = o * cc


def _edge_mlp(edge_attr, edge_weight, ws1, bs1, ws2, bs2, ws3p, bs3p):
    nblk = E // EB
    return pl.pallas_call(
        _edge_kernel,
        grid=(nblk,),
        in_specs=[
            pl.BlockSpec((EB, R), lambda i: (i, 0)),
            pl.BlockSpec((EB, 1), lambda i: (i, 0)),
            pl.BlockSpec((H, R), lambda i: (0, 0)),
            pl.BlockSpec((1, H), lambda i: (0, 0)),
            pl.BlockSpec((2 * H, H), lambda i: (0, 0)),
            pl.BlockSpec((1, 2 * H), lambda i: (0, 0)),
            pl.BlockSpec((NCHUNK, 3 * CW, 2 * H), lambda i: (0, 0, 0)),
            pl.BlockSpec((NCHUNK, 1, 3 * CW), lambda i: (0, 0, 0)),
        ],
        out_specs=pl.BlockSpec((NCHUNK, EB, 3 * CW), lambda i: (0, i, 0)),
        out_shape=jax.ShapeDtypeStruct((NCHUNK, E, 3 * CW), jnp.float32),
    )(edge_attr, edge_weight.reshape(E, 1), ws1, bs1.reshape(1, H), ws2,
      bs2.reshape(1, 2 * H), ws3p, bs3p.reshape(NCHUNK, 1, 3 * CW))


def _conv_kernel(x_ref, p_ref, out_ref):
    out_ref[...] = jnp.dot(x_ref[...], p_ref[...], preferred_element_type=jnp.float32,
                           precision=jax.lax.Precision.HIGHEST)


def _convert(xc, p):
    nblk = N // NB
    return pl.pallas_call(
        _conv_kernel,
        grid=(nblk,),
        in_specs=[
            pl.BlockSpec((NB, 576), lambda i: (i, 0)),
            pl.BlockSpec((576, 576), lambda i: (0, 0)),
        ],
        out_specs=pl.BlockSpec((NB, 576), lambda i: (i, 0)),
        out_shape=jax.ShapeDtypeStruct((N, 576), jnp.float32),
    )(xc, p)


def _prep_kernel(x_ref, gt_ref, gy_ref, t_ref, y_ref, xn_ref, sc_ref):
    x = x_ref[...]
    norm = jnp.zeros((x.shape[0], H), jnp.float32)
    for j in range(9):
        p = x[:, j * H:(j + 1) * H]
        norm += p * p
    inv = 1.0 / (norm + 1.0)
    for j in range(9):
        sc_ref[:, j * H:(j + 1) * H] = x[:, j * H:(j + 1) * H] * inv
    xn = sc_ref[...]
    xn_ref[...] = xn
    t_ref[...] = jnp.dot(xn, gt_ref[...], preferred_element_type=jnp.float32)
    y_ref[...] = jnp.dot(xn, gy_ref[...], preferred_element_type=jnp.float32)


def _node_prep(xc, gt, gy):
    nblk = N // NB
    return pl.pallas_call(
        _prep_kernel,
        grid=(nblk,),
        in_specs=[
            pl.BlockSpec((NB, 576), lambda i: (i, 0)),
            pl.BlockSpec((576, 576), lambda i: (0, 0)),
            pl.BlockSpec((576, 576), lambda i: (0, 0)),
        ],
        out_specs=[
            pl.BlockSpec((NB, 576), lambda i: (i, 0)),
            pl.BlockSpec((NB, 576), lambda i: (i, 0)),
            pl.BlockSpec((NB, 576), lambda i: (i, 0)),
        ],
        out_shape=[
            jax.ShapeDtypeStruct((N, 576), jnp.float32),
            jax.ShapeDtypeStruct((N, 576), jnp.float32),
            jax.ShapeDtypeStruct((N, 576), jnp.float32),
        ],
        scratch_shapes=[pltpu.VMEM((NB, 576), jnp.float32)],
    )(xc, gt, gy)


_SC_TILES = 16
_B = 80
_EPT = E // _SC_TILES
_NB_E = _EPT // _B
_NPT = N // _SC_TILES


def _sc_msg_body(t_hbm, ea_hbm, src_hbm, dst_hbm, zero_hbm, out_hbm,
                 srcv, dstv, recv, eav, acc, sem_g, sem_e, sem_s):
    cid = lax.axis_index("c")
    sid = lax.axis_index("s")
    n0 = sid * _NPT
    e_base = sid * _EPT
    for cqi in range(2):
        cq = cid * 2 + cqi
        pltpu.sync_copy(zero_hbm.at[pl.ds(0, _NPT)], acc.at[pl.ds(n0, _NPT)])
        plsc.subcore_barrier()

        def issue(i, b):
            e0 = e_base + i * _B
            pltpu.sync_copy(src_hbm.at[pl.ds(e0, _B)], srcv.at[b])
            pltpu.sync_copy(dst_hbm.at[pl.ds(e0, _B)], dstv.at[b])
            for j in range(_B // CW):
                d = dstv[b, pl.ds(j * CW, CW)]
                dstv[b, pl.ds(j * CW, CW)] = d * NCHUNK + cq
            gcp = pltpu.async_copy(t_hbm.at[dstv.at[b]], recv.at[b], sem_g)
            ecp = pltpu.async_copy(ea_hbm.at[cq, pl.ds(e0, _B)], eav.at[b], sem_e)
            return gcp, ecp

        issue(0, 0)

        def batch(i, _):
            b = lax.rem(i, 2)
            bn = 1 - b
            pltpu.make_async_copy(t_hbm.at[dstv.at[b]], recv.at[b], sem_g).wait()
            pltpu.make_async_copy(ea_hbm.at[cq, pl.ds(0, _B)], eav.at[b], sem_e).wait()

            @pl.when(i > 0)
            def _():
                pltpu.make_async_copy(recv.at[bn], acc.at[srcv.at[bn]], sem_s).wait()

            @pl.when(i + 1 < _NB_E)
            def _():
                issue(i + 1, bn)

            def edge(e, _):
                w0 = eav[b, e, pl.ds(0, CW)]
                w1 = eav[b, e, pl.ds(CW, CW)]
                w2 = eav[b, e, pl.ds(2 * CW, CW)]
                recv[b, e, pl.ds(0, CW)] = w0 * recv[b, e, pl.ds(0, CW)]
                for j in (1, 2, 3):
                    recv[b, e, pl.ds(j * CW, CW)] = w1 * recv[b, e, pl.ds(j * CW, CW)]
                for j in (4, 5, 6, 7, 8):
                    recv[b, e, pl.ds(j * CW, CW)] = w2 * recv[b, e, pl.ds(j * CW, CW)]
                return 0

            lax.fori_loop(0, _B, edge, 0, unroll=4)
            pltpu.async_copy(recv.at[b], acc.at[srcv.at[b]], sem_s, add=True)
            return 0

        lax.fori_loop(0, _NB_E, batch, 0)
        blast = lax.rem(_NB_E - 1, 2)
        pltpu.make_async_copy(recv.at[blast], acc.at[srcv.at[blast]], sem_s).wait()
        plsc.subcore_barrier()
        pltpu.sync_copy(acc.at[pl.ds(n0, _NPT)], out_hbm.at[pl.ds(n0, _NPT), cq])
        plsc.subcore_barrier()


def _message_pass_sc(t_flat, ea4, src, dst, zeros):
    f = pl.kernel(
        _sc_msg_body,
        out_type=jax.ShapeDtypeStruct((N, NCHUNK, REC), jnp.float32),
        mesh=plsc.VectorSubcoreMesh(core_axis_name="c", subcore_axis_name="s",
                                    num_cores=2, num_subcores=_SC_TILES),
        scratch_types=[
            pltpu.VMEM((2, _B), jnp.int32),
            pltpu.VMEM((2, _B), jnp.int32),
            pltpu.VMEM((2, _B, REC), jnp.float32),
            pltpu.VMEM((2, _B, 3 * CW), jnp.float32),
            pltpu.VMEM_SHARED((N, REC), jnp.float32),
            pltpu.SemaphoreType.DMA,
            pltpu.SemaphoreType.DMA,
            pltpu.SemaphoreType.DMA,
        ],
        compiler_params=pltpu.CompilerParams(use_tc_tiling_on_sc=False),
    )
    return f(t_flat, ea4, src, dst, zeros).reshape(N, 576)


def _update_kernel(xn_ref, y_ref, mc_ref, q_ref, recm_ref, gd_ref, t_ref):
    mp = jnp.dot(mc_ref[...], recm_ref[...], preferred_element_type=jnp.float32)
    y = y_ref[...]
    scale = 1.0 + 0.1 * q_ref[...]

    def pln(a, j):
        return a[:, j * H:(j + 1) * H]

    for r in range(3):
        for c in range(3):
            acc = jnp.zeros((mp.shape[0], H), jnp.float32)
            for k in range(3):
                acc += pln(mp, 3 * r + k) * pln(y, 3 * k + c)
                acc += pln(y, 3 * r + k) * pln(mp, 3 * k + c)
            t_ref[:, (3 * r + c) * H:(3 * r + c + 1) * H] = acc * scale
    dx = jnp.dot(t_ref[...], gd_ref[...], preferred_element_type=jnp.float32)
    xn = xn_ref[...]
    for r in range(3):
        for c in range(3):
            acc = jnp.zeros((dx.shape[0], H), jnp.float32)
            for k in range(3):
                acc += pln(dx, 3 * r + k) * pln(dx, 3 * k + c)
            j = 3 * r + c
            t_ref[:, j * H:(j + 1) * H] = pln(xn, j) + pln(dx, j) + scale * acc


def _node_update(xn, yc, mc, q2, recm, gd):
    nblk = N // NB
    return pl.pallas_call(
        _update_kernel,
        grid=(nblk,),
        in_specs=[
            pl.BlockSpec((NB, 576), lambda i: (i, 0)),
            pl.BlockSpec((NB, 576), lambda i: (i, 0)),
            pl.BlockSpec((NB, 576), lambda i: (i, 0)),
            pl.BlockSpec((NB, 1), lambda i: (i, 0)),
            pl.BlockSpec((576, 576), lambda i: (0, 0)),
            pl.BlockSpec((576, 576), lambda i: (0, 0)),
        ],
        out_specs=pl.BlockSpec((NB, 576), lambda i: (i, 0)),
        out_shape=jax.ShapeDtypeStruct((N, 576), jnp.float32),
    )(xn, yc, mc, q2, recm, gd)


def _post_kernel(x_ref, p2_ref, g_ref, b_ref, wp_ref, lb_ref, out_ref, xhm_ref):
    x = x_ref[...]
    xhm_ref[...] = jnp.dot(x, p2_ref[...], preferred_element_type=jnp.float32,
                           precision=jax.lax.Precision.HIGHEST)

    def pln(j):
        return x[:, j * H:(j + 1) * H]

    lam = (pln(0) + pln(4) + pln(8)) * (1.0 / 3.0)
    a0 = 0.5 * (pln(1) - pln(3)); a1 = 0.5 * (pln(2) - pln(6)); a2 = 0.5 * (pln(5) - pln(7))
    s00 = pln(0) - lam; s01 = 0.5 * (pln(1) + pln(3)); s02 = 0.5 * (pln(2) + pln(6))
    s11 = pln(4) - lam; s12 = 0.5 * (pln(5) + pln(7)); s22 = -s00 - s11
    nI = 3.0 * lam * lam
    nA = 2.0 * (a0 * a0 + a1 * a1 + a2 * a2)
    nS = s00 * s00 + s11 * s11 + s22 * s22 + 2.0 * (s01 * s01 + s02 * s02 + s12 * s12)
    ssum = (jnp.sum(nI, axis=1, keepdims=True) + jnp.sum(nA, axis=1, keepdims=True)
            + jnp.sum(nS, axis=1, keepdims=True))
    ssq = (jnp.sum(nI * nI, axis=1, keepdims=True) + jnp.sum(nA * nA, axis=1, keepdims=True)
           + jnp.sum(nS * nS, axis=1, keepdims=True))
    mean = ssum * (1.0 / (3 * H))
    var = ssq * (1.0 / (3 * H)) - mean * mean
    rstd = jax.lax.rsqrt(var + 1e-5)
    acc = jnp.zeros((x.shape[0], H), jnp.float32)
    for p, npart in enumerate((nI, nA, nS)):
        xi = (npart - mean) * rstd * g_ref[p].reshape(1, H) + b_ref[p].reshape(1, H)
        acc += jnp.dot(xi, wp_ref[p], preferred_element_type=jnp.float32)
    acc = acc + lb_ref[...]
    out_ref[...] = acc * (1.0 / (1.0 + jnp.exp(-acc)))


def _post(xc, p2, ln_g, ln_b, lin_W, lin_b):
    nblk = N // NB
    wp = jnp.stack([lin_W[:, p * H:(p + 1) * H].T for p in range(3)])
    return pl.pallas_call(
        _post_kernel,
        grid=(nblk,),
        in_specs=[
            pl.BlockSpec((NB, 576), lambda i: (i, 0)),
            pl.BlockSpec((576, 576), lambda i: (0, 0)),
            pl.BlockSpec((3, H), lambda i: (0, 0)),
            pl.BlockSpec((3, H), lambda i: (0, 0)),
            pl.BlockSpec((3, H, H), lambda i: (0, 0, 0)),
            pl.BlockSpec((1, H), lambda i: (0, 0)),
        ],
        out_specs=[
            pl.BlockSpec((NB, H), lambda i: (i, 0)),
            pl.BlockSpec((NB, 576), lambda i: (i, 0)),
        ],
        out_shape=[
            jax.ShapeDtypeStruct((N, H), jnp.float32),
            jax.ShapeDtypeStruct((N, 576), jnp.float32),
        ],
    )(xc, p2, ln_g.reshape(3, H), ln_b.reshape(3, H), wp, lin_b.reshape(1, H))


def kernel(X, edge_index, edge_weight, edge_attr, q, ws1, bs1, ws2, bs2, ws3, bs3, wt,
           lin_W, lin_b, ln_g, ln_b):
    from jax._src import config as _jcfg
    with _jcfg.enable_x64(False):
        return _kernel_impl(X, edge_index, edge_weight, edge_attr, q, ws1, bs1, ws2, bs2,
                            ws3, bs3, wt, lin_W, lin_b, ln_g, ln_b)


def _kernel_impl(X, edge_index, edge_weight, edge_attr, q, ws1, bs1, ws2, bs2, ws3, bs3, wt,
                 lin_W, lin_b, ln_g, ln_b):
    L = ws1.shape[0]
    xc = _convert(X.reshape(N, 576), jnp.asarray(_P2.T))
    src = edge_index[0].astype(jnp.int32)
    dst = edge_index[1].astype(jnp.int32)
    zeros = jnp.zeros((_NPT, REC), jnp.float32)
    q2 = q.reshape(N, 1).astype(jnp.float32)
    recm = jnp.asarray(_RecM)
    p2 = jnp.asarray(_P2)

    for l in range(L):
        wst1 = _mix_mats(wt[l, 0], wt[l, 1], wt[l, 2])
        wst2 = _mix_mats(wt[l, 3], wt[l, 4], wt[l, 5])
        gt = _build_GT(wst1)
        gy = _build_GY(wst1)
        gd = _build_GY(wst2)
        ws3p = ws3[l].reshape(NCHUNK, CW, 3, 2 * H).transpose(0, 2, 1, 3).reshape(NCHUNK, 3 * CW, 2 * H)
        bs3p = bs3[l].reshape(NCHUNK, CW, 3).transpose(0, 2, 1).reshape(NCHUNK, 3 * CW)

        ea4 = _edge_mlp(edge_attr, edge_weight.astype(jnp.float32), ws1[l], bs1[l],
                        ws2[l], bs2[l], ws3p, bs3p)
        tc_tab, yc, xn = _node_prep(xc, gt, gy)
        mc = _message_pass_sc(tc_tab.reshape(N * NCHUNK, REC), ea4, src, dst, zeros)
        xc = _node_update(xn, yc, mc, q2, recm, gd)

    post, xhm = _post(xc, p2, ln_g, ln_b, lin_W, lin_b)
    Xout = xhm.reshape(N, H, 3, 3)
    return (Xout, post[:N - 1])


# speedup vs baseline: 1.1737x; 1.0610x over previous
import functools
import math

import jax
import jax.numpy as jnp
import numpy as np
from jax import lax
from jax.experimental import pallas as pl
from jax.experimental.pallas import tpu as pltpu
from jax.experimental.pallas import tpu_sc as plsc

N = 10000
E = 160000
H = 64
R = 32
NCHUNK = 4
CW = H // NCHUNK
REC = 9 * CW
CUTOFF_UPPER = 4.5

NB = 1000
EB = 2000


def _build_D9R9():
    D9 = np.zeros((9, 9), np.float32)
    D9[0, 0] = D9[0, 4] = D9[0, 8] = 1.0 / 3.0
    D9[1, 1], D9[1, 3] = 0.5, -0.5
    D9[2, 2], D9[2, 6] = 0.5, -0.5
    D9[3, 5], D9[3, 7] = 0.5, -0.5
    D9[4, 0], D9[4, 4], D9[4, 8] = 2 / 3, -1 / 3, -1 / 3
    D9[5, 1] = D9[5, 3] = 0.5
    D9[6, 2] = D9[6, 6] = 0.5
    D9[7, 0], D9[7, 4], D9[7, 8] = -1 / 3, 2 / 3, -1 / 3
    D9[8, 5] = D9[8, 7] = 0.5
    R9 = np.zeros((9, 9), np.float32)
    R9[0, 0] = R9[0, 4] = 1
    R9[1, 1] = R9[1, 5] = 1
    R9[2, 2] = R9[2, 6] = 1
    R9[3, 1], R9[3, 5] = -1, 1
    R9[4, 0] = R9[4, 7] = 1
    R9[5, 3] = R9[5, 8] = 1
    R9[6, 2], R9[6, 6] = -1, 1
    R9[7, 3], R9[7, 8] = -1, 1
    R9[8, 0], R9[8, 4], R9[8, 7] = 1, -1, -1
    return D9, R9


_D9, _R9 = _build_D9R9()

_RecM = np.zeros((NCHUNK, 9, CW, 9, H), np.float32)
for _cq in range(NCHUNK):
    for _i in range(9):
        for _c in range(CW):
            _RecM[_cq, _i, _c, :, _cq * CW + _c] = _R9[:, _i]
_RecM = _RecM.reshape(576, 576)

_S64 = np.zeros((576, H), np.float32)
_E64 = np.zeros((H, 576), np.float32)
for _h in range(H):
    for _j in range(9):
        _S64[_h * 9 + _j, _h] = 1.0
        _E64[_h, _h * 9 + _j] = 1.0
_P2 = np.zeros((576, 576), np.float32)
for _h in range(H):
    for _j in range(9):
        _P2[_j * H + _h, _h * 9 + _j] = 1.0
_DEC = np.zeros((9, H, 9, H), np.float32)
for _i in range(9):
    for _j in range(9):
        _DEC[_j, :, _i, :] = _D9[_i, _j] * np.eye(H, dtype=np.float32)
_DEC = _DEC.reshape(9, H, 576).transpose(1, 0, 2).reshape(576, 576)


def _rows_to_hmajor(g):
    return g.reshape(9, H, g.shape[1]).transpose(1, 0, 2).reshape(576, g.shape[1])


def _mix_mats(w0, w1, w2):
    return jnp.stack([w0, w1, w1, w1, w2, w2, w2, w2, w2])


def _build_GY(wst):
    g = jnp.einsum('ij,ki,imh->jhkm', _D9, _R9, wst)
    return g.reshape(576, 576)


def _build_GT(wst):
    g = jnp.einsum('ij,idh->jhid', _D9, wst)
    g = g.reshape(9, H, 9, NCHUNK, CW).transpose(0, 1, 3, 2, 4)
    return g.reshape(576, 576)


def _edge_kernel(attr_ref, ew_ref, w1_ref, b1_ref, w2_ref, b2_ref, w3p_ref, b3p_ref, out_ref):
    def silu(x):
        return x * (1.0 / (1.0 + jnp.exp(-x)))
    a = attr_ref[...]
    h1 = silu(jnp.dot(a, w1_ref[...].T, preferred_element_type=jnp.float32) + b1_ref[...])
    h2 = silu(jnp.dot(h1, w2_ref[...].T, preferred_element_type=jnp.float32) + b2_ref[...])
    ew = ew_ref[...]
    cc = 0.5 * (jnp.cos(ew * (math.pi / CUTOFF_UPPER)) + 1.0)
    cc = jnp.where(ew < CUTOFF_UPPER, cc, 0.0)
    for cq in range(NCHUNK):
        o = silu(jnp.dot(h2, w3p_ref[cq].T, preferred_element_type=jnp.float32) + b3p_ref[cq])
        out_ref[cq] = o * cc


def _edge_mlp(edge_attr, edge_weight, ws1, bs1, ws2, bs2, ws3p, bs3p):
    nblk = E // EB
    return pl.pallas_call(
        _edge_kernel,
        grid=(nblk,),
        in_specs=[
            pl.BlockSpec((EB, R), lambda i: (i, 0)),
            pl.BlockSpec((EB, 1), lambda i: (i, 0)),
            pl.BlockSpec((H, R), lambda i: (0, 0)),
            pl.BlockSpec((1, H), lambda i: (0, 0)),
            pl.BlockSpec((2 * H, H), lambda i: (0, 0)),
            pl.BlockSpec((1, 2 * H), lambda i: (0, 0)),
            pl.BlockSpec((NCHUNK, 3 * CW, 2 * H), lambda i: (0, 0, 0)),
            pl.BlockSpec((NCHUNK, 1, 3 * CW), lambda i: (0, 0, 0)),
        ],
        out_specs=pl.BlockSpec((NCHUNK, EB, 3 * CW), lambda i: (0, i, 0)),
        out_shape=jax.ShapeDtypeStruct((NCHUNK, E, 3 * CW), jnp.float32),
    )(edge_attr, edge_weight.reshape(E, 1), ws1, bs1.reshape(1, H), ws2,
      bs2.reshape(1, 2 * H), ws3p, bs3p.reshape(NCHUNK, 1, 3 * CW))


def _conv_kernel(x_ref, p_ref, out_ref):
    out_ref[...] = jnp.dot(x_ref[...], p_ref[...], preferred_element_type=jnp.float32,
                           precision=jax.lax.Precision.HIGHEST)


def _convert(xc, p):
    nblk = N // NB
    return pl.pallas_call(
        _conv_kernel,
        grid=(nblk,),
        in_specs=[
            pl.BlockSpec((NB, 576), lambda i: (i, 0)),
            pl.BlockSpec((576, 576), lambda i: (0, 0)),
        ],
        out_specs=pl.BlockSpec((NB, 576), lambda i: (i, 0)),
        out_shape=jax.ShapeDtypeStruct((N, 576), jnp.float32),
    )(xc, p)


def _prep_kernel(x_ref, gt_ref, gy_ref, t_ref, y_ref, xn_ref, sc_ref):
    x = x_ref[...]
    norm = jnp.zeros((x.shape[0], H), jnp.float32)
    for j in range(9):
        p = x[:, j * H:(j + 1) * H]
        norm += p * p
    inv = 1.0 / (norm + 1.0)
    for j in range(9):
        sc_ref[:, j * H:(j + 1) * H] = x[:, j * H:(j + 1) * H] * inv
    xn = sc_ref[...]
    xn_ref[...] = xn
    t_ref[...] = jnp.dot(xn, gt_ref[...], preferred_element_type=jnp.float32)
    y_ref[...] = jnp.dot(xn, gy_ref[...], preferred_element_type=jnp.float32)


def _node_prep(xc, gt, gy):
    nblk = N // NB
    return pl.pallas_call(
        _prep_kernel,
        grid=(nblk,),
        in_specs=[
            pl.BlockSpec((NB, 576), lambda i: (i, 0)),
            pl.BlockSpec((576, 576), lambda i: (0, 0)),
            pl.BlockSpec((576, 576), lambda i: (0, 0)),
        ],
        out_specs=[
            pl.BlockSpec((NB, 576), lambda i: (i, 0)),
            pl.BlockSpec((NB, 576), lambda i: (i, 0)),
            pl.BlockSpec((NB, 576), lambda i: (i, 0)),
        ],
        out_shape=[
            jax.ShapeDtypeStruct((N, 576), jnp.float32),
            jax.ShapeDtypeStruct((N, 576), jnp.float32),
            jax.ShapeDtypeStruct((N, 576), jnp.float32),
        ],
        scratch_shapes=[pltpu.VMEM((NB, 576), jnp.float32)],
    )(xc, gt, gy)


_SC_TILES = 16
_B = 80
_EPT = E // _SC_TILES
_NB_E = _EPT // _B
_NPT = N // _SC_TILES


def _sc_msg_body(t_hbm, ea_hbm, src_hbm, dst_hbm, zero_hbm, out_hbm,
                 srcv, dstv, recv, eav, acc, sem_g, sem_e, sem_s):
    cid = lax.axis_index("c")
    sid = lax.axis_index("s")
    n0 = sid * _NPT
    e_base = sid * _EPT
    for cqi in range(2):
        cq = cid * 2 + cqi
        pltpu.sync_copy(zero_hbm.at[pl.ds(0, _NPT)], acc.at[pl.ds(n0, _NPT)])
        plsc.subcore_barrier()

        def issue(i, b):
            e0 = e_base + i * _B
            pltpu.sync_copy(src_hbm.at[pl.ds(e0, _B)], srcv.at[b])
            pltpu.sync_copy(dst_hbm.at[pl.ds(e0, _B)], dstv.at[b])
            for j in range(_B // CW):
                d = dstv[b, pl.ds(j * CW, CW)]
                dstv[b, pl.ds(j * CW, CW)] = d * NCHUNK + cq
            gcp = pltpu.async_copy(t_hbm.at[dstv.at[b]], recv.at[b], sem_g)
            ecp = pltpu.async_copy(ea_hbm.at[cq, pl.ds(e0, _B)], eav.at[b], sem_e)
            return gcp, ecp

        issue(0, 0)

        def batch(i, _):
            b = lax.rem(i, 2)
            bn = 1 - b
            pltpu.make_async_copy(t_hbm.at[dstv.at[b]], recv.at[b], sem_g).wait()
            pltpu.make_async_copy(ea_hbm.at[cq, pl.ds(0, _B)], eav.at[b], sem_e).wait()

            @pl.when(i > 0)
            def _():
                pltpu.make_async_copy(recv.at[bn], acc.at[srcv.at[bn]], sem_s).wait()

            @pl.when(i + 1 < _NB_E)
            def _():
                issue(i + 1, bn)

            def edge(e, _):
                w0 = eav[b, e, pl.ds(0, CW)]
                w1 = eav[b, e, pl.ds(CW, CW)]
                w2 = eav[b, e, pl.ds(2 * CW, CW)]
                recv[b, e, pl.ds(0, CW)] = w0 * recv[b, e, pl.ds(0, CW)]
                for j in (1, 2, 3):
                    recv[b, e, pl.ds(j * CW, CW)] = w1 * recv[b, e, pl.ds(j * CW, CW)]
                for j in (4, 5, 6, 7, 8):
                    recv[b, e, pl.ds(j * CW, CW)] = w2 * recv[b, e, pl.ds(j * CW, CW)]
                return 0

            lax.fori_loop(0, _B, edge, 0, unroll=4)
            pltpu.async_copy(recv.at[b], acc.at[srcv.at[b]], sem_s, add=True)
            return 0

        lax.fori_loop(0, _NB_E, batch, 0)
        blast = lax.rem(_NB_E - 1, 2)
        pltpu.make_async_copy(recv.at[blast], acc.at[srcv.at[blast]], sem_s).wait()
        plsc.subcore_barrier()
        pltpu.sync_copy(acc.at[pl.ds(n0, _NPT)], out_hbm.at[pl.ds(n0, _NPT), cq])
        plsc.subcore_barrier()


def _message_pass_sc(t_flat, ea4, src, dst, zeros):
    f = pl.kernel(
        _sc_msg_body,
        out_type=jax.ShapeDtypeStruct((N, NCHUNK, REC), jnp.float32),
        mesh=plsc.VectorSubcoreMesh(core_axis_name="c", subcore_axis_name="s",
                                    num_cores=2, num_subcores=_SC_TILES),
        scratch_types=[
            pltpu.VMEM((2, _B), jnp.int32),
            pltpu.VMEM((2, _B), jnp.int32),
            pltpu.VMEM((2, _B, REC), jnp.float32),
            pltpu.VMEM((2, _B, 3 * CW), jnp.float32),
            pltpu.VMEM_SHARED((N, REC), jnp.float32),
            pltpu.SemaphoreType.DMA,
            pltpu.SemaphoreType.DMA,
            pltpu.SemaphoreType.DMA,
        ],
        compiler_params=pltpu.CompilerParams(use_tc_tiling_on_sc=False),
    )
    return f(t_flat, ea4, src, dst, zeros).reshape(N, 576)


def _update_kernel(xn_ref, y_ref, mc_ref, q_ref, recm_ref, gd_ref, t_ref):
    mp = jnp.dot(mc_ref[...], recm_ref[...], preferred_element_type=jnp.float32)
    y = y_ref[...]
    scale = 1.0 + 0.1 * q_ref[...]

    def pln(a, j):
        return a[:, j * H:(j + 1) * H]

    for r in range(3):
        for c in range(3):
            acc = jnp.zeros((mp.shape[0], H), jnp.float32)
            for k in range(3):
                acc += pln(mp, 3 * r + k) * pln(y, 3 * k + c)
                acc += pln(y, 3 * r + k) * pln(mp, 3 * k + c)
            t_ref[:, (3 * r + c) * H:(3 * r + c + 1) * H] = acc * scale
    dx = jnp.dot(t_ref[...], gd_ref[...], preferred_element_type=jnp.float32)
    xn = xn_ref[...]
    for r in range(3):
        for c in range(3):
            acc = jnp.zeros((dx.shape[0], H), jnp.float32)
            for k in range(3):
                acc += pln(dx, 3 * r + k) * pln(dx, 3 * k + c)
            j = 3 * r + c
            t_ref[:, j * H:(j + 1) * H] = pln(xn, j) + pln(dx, j) + scale * acc


def _node_update(xn, yc, mc, q2, recm, gd):
    nblk = N // NB
    return pl.pallas_call(
        _update_kernel,
        grid=(nblk,),
        in_specs=[
            pl.BlockSpec((NB, 576), lambda i: (i, 0)),
            pl.BlockSpec((NB, 576), lambda i: (i, 0)),
            pl.BlockSpec((NB, 576), lambda i: (i, 0)),
            pl.BlockSpec((NB, 1), lambda i: (i, 0)),
            pl.BlockSpec((576, 576), lambda i: (0, 0)),
            pl.BlockSpec((576, 576), lambda i: (0, 0)),
        ],
        out_specs=pl.BlockSpec((NB, 576), lambda i: (i, 0)),
        out_shape=jax.ShapeDtypeStruct((N, 576), jnp.float32),
    )(xn, yc, mc, q2, recm, gd)


def _post_kernel(x_ref, g_ref, b_ref, wp_ref, lb_ref, out_ref):
    x = x_ref[...]

    def pln(j):
        return x[:, j * H:(j + 1) * H]

    lam = (pln(0) + pln(4) + pln(8)) * (1.0 / 3.0)
    a0 = 0.5 * (pln(1) - pln(3)); a1 = 0.5 * (pln(2) - pln(6)); a2 = 0.5 * (pln(5) - pln(7))
    s00 = pln(0) - lam; s01 = 0.5 * (pln(1) + pln(3)); s02 = 0.5 * (pln(2) + pln(6))
    s11 = pln(4) - lam; s12 = 0.5 * (pln(5) + pln(7)); s22 = -s00 - s11
    nI = 3.0 * lam * lam
    nA = 2.0 * (a0 * a0 + a1 * a1 + a2 * a2)
    nS = s00 * s00 + s11 * s11 + s22 * s22 + 2.0 * (s01 * s01 + s02 * s02 + s12 * s12)
    ssum = (jnp.sum(nI, axis=1, keepdims=True) + jnp.sum(nA, axis=1, keepdims=True)
            + jnp.sum(nS, axis=1, keepdims=True))
    ssq = (jnp.sum(nI * nI, axis=1, keepdims=True) + jnp.sum(nA * nA, axis=1, keepdims=True)
           + jnp.sum(nS * nS, axis=1, keepdims=True))
    mean = ssum * (1.0 / (3 * H))
    var = ssq * (1.0 / (3 * H)) - mean * mean
    rstd = jax.lax.rsqrt(var + 1e-5)
    acc = jnp.zeros((x.shape[0], H), jnp.float32)
    for p, npart in enumerate((nI, nA, nS)):
        xi = (npart - mean) * rstd * g_ref[p].reshape(1, H) + b_ref[p].reshape(1, H)
        acc += jnp.dot(xi, wp_ref[p], preferred_element_type=jnp.float32)
    acc = acc + lb_ref[...]
    out_ref[...] = acc * (1.0 / (1.0 + jnp.exp(-acc)))


def _post(xc, ln_g, ln_b, lin_W, lin_b):
    nblk = N // NB
    wp = jnp.stack([lin_W[:, p * H:(p + 1) * H].T for p in range(3)])
    return pl.pallas_call(
        _post_kernel,
        grid=(nblk,),
        in_specs=[
            pl.BlockSpec((NB, 576), lambda i: (i, 0)),
            pl.BlockSpec((3, H), lambda i: (0, 0)),
            pl.BlockSpec((3, H), lambda i: (0, 0)),
            pl.BlockSpec((3, H, H), lambda i: (0, 0, 0)),
            pl.BlockSpec((1, H), lambda i: (0, 0)),
        ],
        out_specs=pl.BlockSpec((NB, H), lambda i: (i, 0)),
        out_shape=jax.ShapeDtypeStruct((N, H), jnp.float32),
    )(xc, ln_g.reshape(3, H), ln_b.reshape(3, H), wp, lin_b.reshape(1, H))


def kernel(X, edge_index, edge_weight, edge_attr, q, ws1, bs1, ws2, bs2, ws3, bs3, wt,
           lin_W, lin_b, ln_g, ln_b):
    from jax._src import config as _jcfg
    with _jcfg.enable_x64(False):
        return _kernel_impl(X, edge_index, edge_weight, edge_attr, q, ws1, bs1, ws2, bs2,
                            ws3, bs3, wt, lin_W, lin_b, ln_g, ln_b)


def _kernel_impl(X, edge_index, edge_weight, edge_attr, q, ws1, bs1, ws2, bs2, ws3, bs3, wt,
                 lin_W, lin_b, ln_g, ln_b):
    L = ws1.shape[0]
    xc = jnp.transpose(X, (0, 2, 3, 1)).reshape(N, 576)
    src = edge_index[0].astype(jnp.int32)
    dst = edge_index[1].astype(jnp.int32)
    zeros = jnp.zeros((_NPT, REC), jnp.float32)
    q2 = q.reshape(N, 1).astype(jnp.float32)
    recm = jnp.asarray(_RecM)
    p2 = jnp.asarray(_P2)

    for l in range(L):
        wst1 = _mix_mats(wt[l, 0], wt[l, 1], wt[l, 2])
        wst2 = _mix_mats(wt[l, 3], wt[l, 4], wt[l, 5])
        gt = _build_GT(wst1)
        gy = _build_GY(wst1)
        gd = _build_GY(wst2)
        ws3p = ws3[l].reshape(NCHUNK, CW, 3, 2 * H).transpose(0, 2, 1, 3).reshape(NCHUNK, 3 * CW, 2 * H)
        bs3p = bs3[l].reshape(NCHUNK, CW, 3).transpose(0, 2, 1).reshape(NCHUNK, 3 * CW)

        ea4 = _edge_mlp(edge_attr, edge_weight.astype(jnp.float32), ws1[l], bs1[l],
                        ws2[l], bs2[l], ws3p, bs3p)
        tc_tab, yc, xn = _node_prep(xc, gt, gy)
        mc = _message_pass_sc(tc_tab.reshape(N * NCHUNK, REC), ea4, src, dst, zeros)
        xc = _node_update(xn, yc, mc, q2, recm, gd)

    post = _post(xc, ln_g, ln_b, lin_W, lin_b)
    Xout = xc.reshape(N, 3, 3, H).transpose(0, 3, 1, 2)
    return (Xout, post[:N - 1])
